# baseline (speedup 1.0000x reference)
import jax
import jax.numpy as jnp
from jax import lax
from jax.experimental import pallas as pl
from jax.experimental.pallas import tpu as pltpu
from jax.experimental.pallas import tpu_sc as plsc

N = 10000
E = 320000
D = 128
A = 32

ROWS_PT = N // 16
CHUNK = 128
EROW = E // CHUNK
NCK_LO = EROW // 16
NCK_HI = NCK_LO + 1
NCK_REM = EROW % 16

DEPTH = 6
NBUF = 8


def _mp_round(nck, table_of, src_v, dst_v, msg_v, acc_s, sem_g, sem_s):
    for d in range(DEPTH):
        pltpu.async_copy(table_of(src_v.at[d]), msg_v.at[d], sem_g)

    def chunk(j, carry):
        b = lax.rem(j, NBUF)
        pltpu.make_async_copy(table_of(src_v.at[j]), msg_v.at[b], sem_g).wait()

        @pl.when(j + DEPTH < nck)
        def _():
            pltpu.async_copy(table_of(src_v.at[j + DEPTH]),
                             msg_v.at[lax.rem(j + DEPTH, NBUF)], sem_g)

        pltpu.async_copy(msg_v.at[b], acc_s.at[dst_v.at[j]], sem_s, add=True)

        @pl.when(j > 0)
        def _():
            pltpu.make_async_copy(
                msg_v.at[b], acc_s.at[dst_v.at[j]], sem_s).wait()

        return carry

    lax.fori_loop(0, nck, chunk, 0)
    pltpu.make_async_copy(msg_v.at[0], acc_s.at[dst_v.at[0]], sem_s).wait()


def _sc_body(table1, d1h, srcp, dstp, srcl, dstl, zeros_h, h1_out, acc2_out,
             src_v, dst_v, msg_v, row_v, rowd_v, acc_s, sem_g, sem_s):
    cid = lax.axis_index("c")
    sid = lax.axis_index("s")
    lrow = sid * ROWS_PT
    rows = pl.ds(lrow, ROWS_PT)

    nck = jnp.where(sid < NCK_REM, NCK_HI, NCK_LO)
    start = sid * NCK_LO + jnp.minimum(sid, NCK_REM)
    lo = pl.ds(start, NCK_LO)
    hi1 = pl.ds(start + NCK_LO, 1)

    @pl.when(cid == 0)
    def _():
        pltpu.sync_copy(srcp.at[lo], src_v.at[pl.ds(0, NCK_LO)])
        pltpu.sync_copy(dstp.at[lo], dst_v.at[pl.ds(0, NCK_LO)])

        @pl.when(sid < NCK_REM)
        def _():
            pltpu.sync_copy(srcp.at[hi1], src_v.at[pl.ds(NCK_LO, 1)])
            pltpu.sync_copy(dstp.at[hi1], dst_v.at[pl.ds(NCK_LO, 1)])

    @pl.when(cid == 1)
    def _():
        pltpu.sync_copy(srcl.at[lo], src_v.at[pl.ds(0, NCK_LO)])
        pltpu.sync_copy(dstl.at[lo], dst_v.at[pl.ds(0, NCK_LO)])

        @pl.when(sid < NCK_REM)
        def _():
            pltpu.sync_copy(srcl.at[hi1], src_v.at[pl.ds(NCK_LO, 1)])
            pltpu.sync_copy(dstl.at[hi1], dst_v.at[pl.ds(NCK_LO, 1)])

    pltpu.sync_copy(zeros_h, row_v)
    pltpu.sync_copy(row_v, acc_s.at[rows])
    plsc.subcore_barrier()

    _mp_round(nck, lambda idx: table1.at[cid].at[idx],
              src_v, dst_v, msg_v, acc_s, sem_g, sem_s)
    plsc.subcore_barrier()

    pltpu.sync_copy(acc_s.at[rows], row_v)
    pltpu.sync_copy(d1h.at[cid].at[rows], rowd_v)

    def act(i, carry):
        row_v[i] = jnp.maximum(row_v[i] + rowd_v[i], 0.0)
        return carry

    lax.fori_loop(0, ROWS_PT, act, 0)
    pltpu.sync_copy(row_v, h1_out.at[cid].at[rows])
    pltpu.sync_copy(zeros_h, rowd_v)
    pltpu.sync_copy(rowd_v, acc_s.at[rows])
    plsc.subcore_barrier()

    _mp_round(nck, lambda idx: h1_out.at[cid].at[idx],
              src_v, dst_v, msg_v, acc_s, sem_g, sem_s)
    plsc.subcore_barrier()

    pltpu.sync_copy(acc_s.at[rows], row_v)
    pltpu.sync_copy(row_v, acc2_out.at[cid].at[rows])


@jax.jit
def _sc_mp(table1, d1, srcp, dstp, srcl, dstl, zeros_rows):
    mesh = plsc.VectorSubcoreMesh(core_axis_name="c", subcore_axis_name="s")
    return pl.kernel(
        _sc_body,
        out_type=(jax.ShapeDtypeStruct((2, N, 16), jnp.float32),
                  jax.ShapeDtypeStruct((2, N, 16), jnp.float32)),
        mesh=mesh,
        scratch_types=[
            pltpu.VMEM((NCK_HI, CHUNK), jnp.int32),
            pltpu.VMEM((NCK_HI, CHUNK), jnp.int32),
            pltpu.VMEM((NBUF, CHUNK, 16), jnp.float32),
            pltpu.VMEM((ROWS_PT, 16), jnp.float32),
            pltpu.VMEM((ROWS_PT, 16), jnp.float32),
            pltpu.VMEM_SHARED((N, 16), jnp.float32),
            pltpu.SemaphoreType.DMA,
            pltpu.SemaphoreType.DMA,
        ],
        compiler_params=pltpu.CompilerParams(use_tc_tiling_on_sc=False),
    )(table1, d1, srcp, dstp, srcl, dstl, zeros_rows)


BLK = 5000


def _pre_body(px, lx, wrp, wsp, bp, wrl, wsl, bl, m1, d1):
    xp = px[...]
    xl = lx[...]
    m1[0] = jnp.dot(xp, wrp[...], preferred_element_type=jnp.float32)
    m1[1] = jnp.dot(xl, wrl[...], preferred_element_type=jnp.float32)
    d1[0] = jnp.dot(xp, wsp[...], preferred_element_type=jnp.float32) + bp[...]
    d1[1] = jnp.dot(xl, wsl[...], preferred_element_type=jnp.float32) + bl[...]


def _head_body(a2p, h1p, a2l, h1l, act,
               wrp2, wsp2, bp2, wrl2, wsl2, bl2,
               pin_w, pin_b, ph_w, ph_b, po_w, po_b, out):
    p2 = (jnp.dot(a2p[0], wrp2[...], preferred_element_type=jnp.float32)
          + jnp.dot(h1p[0], wsp2[...], preferred_element_type=jnp.float32)
          + bp2[...])
    l2 = (jnp.dot(a2l[0], wrl2[...], preferred_element_type=jnp.float32)
          + jnp.dot(h1l[0], wsl2[...], preferred_element_type=jnp.float32)
          + bl2[...])
    mol = jnp.concatenate([p2, l2], axis=1)
    fp = jnp.maximum(
        jnp.dot(mol, pin_w[...], preferred_element_type=jnp.float32) + pin_b[...],
        0.0)
    pol = (jnp.dot(jnp.concatenate([fp, act[...]], axis=1), ph_w[...],
                   preferred_element_type=jnp.float32) + ph_b[...])
    out[...] = (jnp.dot(jnp.maximum(pol, 0.0), po_w[...],
                        preferred_element_type=jnp.float32) + po_b[...])


def _row_spec(block, width):
    return pl.BlockSpec((block, width), lambda i: (i, 0))


def _plane_spec(plane):
    return pl.BlockSpec((1, BLK, 16), lambda i, p=plane: (p, i, 0))


def _full_spec():
    return pl.BlockSpec()


def kernel(protein_x, ligand_x, action,
           pr_in_Wr, pr_in_Ws, pr_in_b, pr_out_Wr, pr_out_Ws, pr_out_b,
           lg_in_Wr, lg_in_Ws, lg_in_b, lg_out_Wr, lg_out_Ws, lg_out_b,
           pin_W, pin_b, ph_W, ph_b, po_W, po_b,
           protein_edge_index, ligand_edge_index):
    f32 = jnp.float32
    grid = (N // BLK,)

    srcp = protein_edge_index[0].reshape(EROW, CHUNK)
    dstp = protein_edge_index[1].reshape(EROW, CHUNK)
    srcl = ligand_edge_index[0].reshape(EROW, CHUNK)
    dstl = ligand_edge_index[1].reshape(EROW, CHUNK)
    zeros_rows = jnp.zeros((ROWS_PT, 16), f32)

    table1, d1 = pl.pallas_call(
        _pre_body,
        grid=grid,
        in_specs=[_row_spec(BLK, D), _row_spec(BLK, D)] + [_full_spec()] * 6,
        out_specs=[pl.BlockSpec((2, BLK, 16), lambda i: (0, i, 0))] * 2,
        out_shape=[jax.ShapeDtypeStruct((2, N, 16), f32)] * 2,
    )(protein_x, ligand_x,
      pr_in_Wr.T, pr_in_Ws.T, pr_in_b.reshape(1, 16),
      lg_in_Wr.T, lg_in_Ws.T, lg_in_b.reshape(1, 16))

    h1, acc2 = _sc_mp(table1, d1, srcp, dstp, srcl, dstl, zeros_rows)

    out = pl.pallas_call(
        _head_body,
        grid=grid,
        in_specs=[_plane_spec(0), _plane_spec(0), _plane_spec(1),
                  _plane_spec(1), _row_spec(BLK, A)] + [_full_spec()] * 12,
        out_specs=_row_spec(BLK, 1),
        out_shape=jax.ShapeDtypeStruct((N, 1), f32),
    )(acc2, h1, acc2, h1, action,
      pr_out_Wr.T, pr_out_Ws.T, pr_out_b.reshape(1, 50),
      lg_out_Wr.T, lg_out_Ws.T, lg_out_b.reshape(1, 50),
      pin_W.T, pin_b.reshape(1, 60), ph_W.T, ph_b.reshape(1, 10),
      po_W.T, po_b.reshape(1, 1))
    return out


# speedup vs baseline: 1.1171x; 1.1171x over previous
import jax
import jax.numpy as jnp
from jax import lax
from jax.experimental import pallas as pl
from jax.experimental.pallas import tpu as pltpu
from jax.experimental.pallas import tpu_sc as plsc

N = 10000
E = 320000
D = 128
A = 32

ROWS_PT = N // 16
CHUNK = 125
NCHUNK = 160
EROW = E // CHUNK

DEPTH = 6
NBUF = 8


def _mp_round(nck, table_of, src_v, dst_v, msg_v, acc_s, sem_g, sem_s):
    for d in range(DEPTH):
        pltpu.async_copy(table_of(src_v.at[d]), msg_v.at[d], sem_g)

    def chunk(j, carry):
        b = lax.rem(j, NBUF)
        pltpu.make_async_copy(table_of(src_v.at[j]), msg_v.at[b], sem_g).wait()

        @pl.when(j + DEPTH < nck)
        def _():
            pltpu.async_copy(table_of(src_v.at[j + DEPTH]),
                             msg_v.at[lax.rem(j + DEPTH, NBUF)], sem_g)

        pltpu.async_copy(msg_v.at[b], acc_s.at[dst_v.at[j]], sem_s, add=True)

        @pl.when(j > 0)
        def _():
            pltpu.make_async_copy(
                msg_v.at[b], acc_s.at[dst_v.at[j]], sem_s).wait()

        return carry

    lax.fori_loop(0, nck, chunk, 0)
    pltpu.make_async_copy(msg_v.at[0], acc_s.at[dst_v.at[0]], sem_s).wait()


def _sc_body(table1, d1h, pei, lei, zeros_h, h1_out, acc2_out,
             src_v, dst_v, msg_v, row_v, rowd_v, acc_s, tbl2_s, sem_g, sem_s):
    cid = lax.axis_index("c")
    sid = lax.axis_index("s")
    lrow = sid * ROWS_PT
    rows = pl.ds(lrow, ROWS_PT)
    nck = NCHUNK
    echunks = pl.ds(sid * NCHUNK, NCHUNK)

    @pl.when(cid == 0)
    def _():
        pltpu.sync_copy(pei.at[0].at[echunks], src_v)
        pltpu.sync_copy(pei.at[1].at[echunks], dst_v)

    @pl.when(cid == 1)
    def _():
        pltpu.sync_copy(lei.at[0].at[echunks], src_v)
        pltpu.sync_copy(lei.at[1].at[echunks], dst_v)

    pltpu.sync_copy(zeros_h, row_v)
    pltpu.sync_copy(row_v, acc_s.at[rows])
    plsc.subcore_barrier()

    _mp_round(nck, lambda idx: table1.at[cid].at[idx],
              src_v, dst_v, msg_v, acc_s, sem_g, sem_s)
    plsc.subcore_barrier()

    pltpu.sync_copy(acc_s.at[rows], row_v)
    pltpu.sync_copy(d1h.at[cid].at[rows], rowd_v)

    def act(i, carry):
        row_v[i] = jnp.maximum(row_v[i] + rowd_v[i], 0.0)
        return carry

    lax.fori_loop(0, ROWS_PT, act, 0)
    pltpu.sync_copy(row_v, h1_out.at[cid].at[rows])
    pltpu.sync_copy(row_v, tbl2_s.at[rows])
    pltpu.sync_copy(zeros_h, rowd_v)
    pltpu.sync_copy(rowd_v, acc_s.at[rows])
    plsc.subcore_barrier()

    _mp_round(nck, lambda idx: tbl2_s.at[idx],
              src_v, dst_v, msg_v, acc_s, sem_g, sem_s)
    plsc.subcore_barrier()

    pltpu.sync_copy(acc_s.at[rows], row_v)
    pltpu.sync_copy(row_v, acc2_out.at[cid].at[rows])


@jax.jit
def _sc_mp(table1, d1, pei, lei, zeros_rows):
    mesh = plsc.VectorSubcoreMesh(core_axis_name="c", subcore_axis_name="s")
    return pl.kernel(
        _sc_body,
        out_type=(jax.ShapeDtypeStruct((2, N, 16), jnp.float32),
                  jax.ShapeDtypeStruct((2, N, 16), jnp.float32)),
        mesh=mesh,
        scratch_types=[
            pltpu.VMEM((NCHUNK, CHUNK), jnp.int32),
            pltpu.VMEM((NCHUNK, CHUNK), jnp.int32),
            pltpu.VMEM((NBUF, CHUNK, 16), jnp.float32),
            pltpu.VMEM((ROWS_PT, 16), jnp.float32),
            pltpu.VMEM((ROWS_PT, 16), jnp.float32),
            pltpu.VMEM_SHARED((N, 16), jnp.float32),
            pltpu.VMEM_SHARED((N, 16), jnp.float32),
            pltpu.SemaphoreType.DMA,
            pltpu.SemaphoreType.DMA,
        ],
        compiler_params=pltpu.CompilerParams(use_tc_tiling_on_sc=False),
    )(table1, d1, pei, lei, zeros_rows)


BLK = 5000


def _pre_body(px, lx, wrp, wsp, bp, wrl, wsl, bl, m1, d1):
    xp = px[...]
    xl = lx[...]
    m1[0] = jnp.dot(xp, wrp[...], preferred_element_type=jnp.float32)
    m1[1] = jnp.dot(xl, wrl[...], preferred_element_type=jnp.float32)
    d1[0] = jnp.dot(xp, wsp[...], preferred_element_type=jnp.float32) + bp[...]
    d1[1] = jnp.dot(xl, wsl[...], preferred_element_type=jnp.float32) + bl[...]


def _head_body(a2p, h1p, a2l, h1l, act,
               wrp2, wsp2, bp2, wrl2, wsl2, bl2,
               pin_w, pin_b, ph_w, ph_b, po_w, po_b, out):
    p2 = (jnp.dot(a2p[0], wrp2[...], preferred_element_type=jnp.float32)
          + jnp.dot(h1p[0], wsp2[...], preferred_element_type=jnp.float32)
          + bp2[...])
    l2 = (jnp.dot(a2l[0], wrl2[...], preferred_element_type=jnp.float32)
          + jnp.dot(h1l[0], wsl2[...], preferred_element_type=jnp.float32)
          + bl2[...])
    mol = jnp.concatenate([p2, l2], axis=1)
    fp = jnp.maximum(
        jnp.dot(mol, pin_w[...], preferred_element_type=jnp.float32) + pin_b[...],
        0.0)
    pol = (jnp.dot(jnp.concatenate([fp, act[...]], axis=1), ph_w[...],
                   preferred_element_type=jnp.float32) + ph_b[...])
    out[...] = (jnp.dot(jnp.maximum(pol, 0.0), po_w[...],
                        preferred_element_type=jnp.float32) + po_b[...])


def _row_spec(block, width):
    return pl.BlockSpec((block, width), lambda i: (i, 0))


def _plane_spec(plane):
    return pl.BlockSpec((1, BLK, 16), lambda i, p=plane: (p, i, 0))


def _full_spec():
    return pl.BlockSpec()


def kernel(protein_x, ligand_x, action,
           pr_in_Wr, pr_in_Ws, pr_in_b, pr_out_Wr, pr_out_Ws, pr_out_b,
           lg_in_Wr, lg_in_Ws, lg_in_b, lg_out_Wr, lg_out_Ws, lg_out_b,
           pin_W, pin_b, ph_W, ph_b, po_W, po_b,
           protein_edge_index, ligand_edge_index):
    f32 = jnp.float32
    grid = (N // BLK,)

    pei = protein_edge_index.reshape(2, EROW, CHUNK)
    lei = ligand_edge_index.reshape(2, EROW, CHUNK)
    zeros_rows = jnp.zeros((ROWS_PT, 16), f32)

    table1, d1 = pl.pallas_call(
        _pre_body,
        grid=grid,
        in_specs=[_row_spec(BLK, D), _row_spec(BLK, D)] + [_full_spec()] * 6,
        out_specs=[pl.BlockSpec((2, BLK, 16), lambda i: (0, i, 0))] * 2,
        out_shape=[jax.ShapeDtypeStruct((2, N, 16), f32)] * 2,
    )(protein_x, ligand_x,
      pr_in_Wr.T, pr_in_Ws.T, pr_in_b.reshape(1, 16),
      lg_in_Wr.T, lg_in_Ws.T, lg_in_b.reshape(1, 16))

    h1, acc2 = _sc_mp(table1, d1, pei, lei, zeros_rows)

    out = pl.pallas_call(
        _head_body,
        grid=grid,
        in_specs=[_plane_spec(0), _plane_spec(0), _plane_spec(1),
                  _plane_spec(1), _row_spec(BLK, A)] + [_full_spec()] * 12,
        out_specs=_row_spec(BLK, 1),
        out_shape=jax.ShapeDtypeStruct((N, 1), f32),
    )(acc2, h1, acc2, h1, action,
      pr_out_Wr.T, pr_out_Ws.T, pr_out_b.reshape(1, 50),
      lg_out_Wr.T, lg_out_Ws.T, lg_out_b.reshape(1, 50),
      pin_W.T, pin_b.reshape(1, 60), ph_W.T, ph_b.reshape(1, 10),
      po_W.T, po_b.reshape(1, 1))
    return out
